# Initial kernel scaffold; baseline (speedup 1.0000x reference)
import jax
import jax.numpy as jnp
from jax.experimental import pallas as pl


def kernel(flow_fingerprint, x, edge_attr, edge_index, batch, params):
    raise NotImplementedError("write your pallas kernel here")


# speedup vs baseline: 1.0269x; 1.0269x over previous
import jax
import jax.numpy as jnp
from jax.experimental import pallas as pl

H = 128
L = 3


def _ln(x, g, b):
    m = jnp.mean(x, axis=-1, keepdims=True)
    v = jnp.var(x, axis=-1, keepdims=True)
    return (x - m) / jnp.sqrt(v + 1e-5) * g + b


def _lstm(xseq, lstm_params):
    h_seq = xseq
    hT = None
    for lp in lstm_params:
        Wih, Whh, b = lp['Wih'], lp['Whh'], lp['b']
        B = h_seq.shape[0]
        Hh = Whh.shape[1]
        def step(carry, xt):
            h, c = carry
            gates = xt @ Wih.T + h @ Whh.T + b
            i, f, g, o = jnp.split(gates, 4, axis=-1)
            i = jax.nn.sigmoid(i)
            f = jax.nn.sigmoid(f)
            g = jnp.tanh(g)
            o = jax.nn.sigmoid(o)
            c = f * c + i * g
            h = o * jnp.tanh(c)
            return (h, c), h
        init = (jnp.zeros((B, Hh), jnp.float32), jnp.zeros((B, Hh), jnp.float32))
        (hT, _), ys = jax.lax.scan(step, init, jnp.swapaxes(h_seq, 0, 1))
        h_seq = jnp.swapaxes(ys, 0, 1)
    return hT


def _copy_kernel(x_ref, o_ref):
    o_ref[...] = x_ref[...]


def kernel(flow_fingerprint, x, edge_attr, edge_index, batch, params):
    p = params
    h_flow = _lstm(flow_fingerprint, p['lstm'])
    film = jax.nn.relu(h_flow @ p['film_W1'] + p['film_b1']) @ p['film_W2'] + p['film_b2']
    film = film.reshape(-1, L, 2, H)
    gamma = 1.0 + film[0, :, 0, :]
    beta = film[0, :, 1, :]
    src = edge_index[0]
    dst = edge_index[1]

    h = x @ p['embed_W'] + p['embed_b']
    for l in range(L):
        cp = p['convs'][l]
        hW = h @ cp['Wmsg']
        eW = edge_attr @ cp['Wedge'] + cp['b']
        msg = jax.nn.relu(jnp.take(hW, src, axis=0) + eW)
        agg = jax.ops.segment_sum(msg, dst, num_segments=h.shape[0])
        h = _ln(h + agg, cp['ln_g'], cp['ln_b'])
        h = gamma[l][None, :] * h + beta[l][None, :]

    W1 = p['head_W1']
    As = h @ W1[:H]
    Bd = h @ W1[H:2 * H]
    C = edge_attr @ W1[2 * H:] + p['head_b1']
    pre = jax.nn.relu(jnp.take(As, src, axis=0) + jnp.take(Bd, dst, axis=0) + C)
    edge_logits = (pre @ p['head_W2'] + p['head_b2'])[:, 0]

    h_graph = jnp.mean(h, axis=0, keepdims=True)
    state = jnp.concatenate([h_flow, h_graph], axis=1)
    value = (jax.nn.relu(state @ p['critic_W1'] + p['critic_b1']) @ p['critic_W2'] + p['critic_b2'])[:, 0]

    value = pl.pallas_call(
        _copy_kernel,
        out_shape=jax.ShapeDtypeStruct(value.shape, value.dtype),
    )(value)
    return edge_logits, value


# speedup vs baseline: 1.3221x; 1.2875x over previous
import functools

import jax
import jax.numpy as jnp
from jax import lax
from jax.experimental import pallas as pl
from jax.experimental.pallas import tpu as pltpu
from jax.experimental.pallas import tpu_sc as plsc

N = 10000
E = 320000
F = 128
L = 3
NC = 2
NS = 16
NW = NC * NS
LANES = 16
CPT = F // NW
CHUNK = 2560
NCHUNK = E // CHUNK
GROUPS = CHUNK // LANES


def _msgpass_body(hw_ref, ew_ref, src_ref, dst_ref, out_ref,
                  table, agg, src_buf, dst_buf, ew_buf):
    c = lax.axis_index("core")
    s = lax.axis_index("sub")
    w = c * NS + s
    col = w * CPT
    half = w % 2

    pltpu.sync_copy(hw_ref.at[pl.ds(col * N, CPT * N)], table)

    zeros = jnp.zeros((LANES,), jnp.float32)
    def _zero(i, carry):
        table_dummy = carry
        agg[pl.ds(i * LANES, LANES)] = zeros
        return table_dummy
    lax.fori_loop(0, CPT * N // LANES, _zero, 0)

    def _chunk(k, carry):
        base = k * CHUNK
        pltpu.sync_copy(src_ref.at[pl.ds(base, CHUNK)], src_buf)
        pltpu.sync_copy(dst_ref.at[pl.ds(base, CHUNK)], dst_buf)
        pltpu.sync_copy(ew_ref.at[k, w // 2], ew_buf)

        def _group(g, carry2):
            si = src_buf[pl.ds(g * LANES, LANES)]
            di = dst_buf[pl.ds(g * LANES, LANES)]
            for cc in range(CPT):
                gath = plsc.load_gather(table, [si + jnp.int32(cc * N)])
                ew = ew_buf[half * CPT + cc, pl.ds(g * LANES, LANES)]
                m = jnp.maximum(gath + ew, 0.0)
                plsc.addupdate_scatter(agg, [di + jnp.int32(cc * N)], m)
            return carry2
        lax.fori_loop(0, GROUPS, _group, 0)
        return carry
    lax.fori_loop(0, NCHUNK, _chunk, 0)

    pltpu.sync_copy(agg, out_ref.at[pl.ds(col * N, CPT * N)])


@jax.jit
def _msgpass(hw_flat, ew4, src, dst):
    mesh = plsc.VectorSubcoreMesh(core_axis_name="core", subcore_axis_name="sub")
    return pl.kernel(
        _msgpass_body,
        out_type=jax.ShapeDtypeStruct((F * N,), jnp.float32),
        mesh=mesh,
        compiler_params=pltpu.CompilerParams(needs_layout_passes=False),
        scratch_types=[
            pltpu.VMEM((CPT * N,), jnp.float32),
            pltpu.VMEM((CPT * N,), jnp.float32),
            pltpu.VMEM((CHUNK,), jnp.int32),
            pltpu.VMEM((CHUNK,), jnp.int32),
            pltpu.VMEM((8, CHUNK), jnp.float32),
        ],
    )(hw_flat, ew4, src, dst)


def _ln(x, g, b):
    m = jnp.mean(x, axis=-1, keepdims=True)
    v = jnp.var(x, axis=-1, keepdims=True)
    return (x - m) / jnp.sqrt(v + 1e-5) * g + b


def _lstm(xseq, lstm_params):
    h_seq = xseq
    hT = None
    for lp in lstm_params:
        Wih, Whh, b = lp['Wih'], lp['Whh'], lp['b']
        B = h_seq.shape[0]
        Hh = Whh.shape[1]
        def step(carry, xt):
            h, cst = carry
            gates = xt @ Wih.T + h @ Whh.T + b
            i, f, g, o = jnp.split(gates, 4, axis=-1)
            i = jax.nn.sigmoid(i)
            f = jax.nn.sigmoid(f)
            g = jnp.tanh(g)
            o = jax.nn.sigmoid(o)
            cst = f * cst + i * g
            h = o * jnp.tanh(cst)
            return (h, cst), h
        init = (jnp.zeros((B, Hh), jnp.float32), jnp.zeros((B, Hh), jnp.float32))
        (hT, _), ys = lax.scan(step, init, jnp.swapaxes(h_seq, 0, 1))
        h_seq = jnp.swapaxes(ys, 0, 1)
    return hT


def _to_group_layout(ew_t):
    return ew_t.reshape(F // 8, 8, NCHUNK, CHUNK).transpose(2, 0, 1, 3)


def kernel(flow_fingerprint, x, edge_attr, edge_index, batch, params):
    p = params
    h_flow = _lstm(flow_fingerprint, p['lstm'])
    film = jax.nn.relu(h_flow @ p['film_W1'] + p['film_b1']) @ p['film_W2'] + p['film_b2']
    film = film.reshape(-1, L, 2, F)
    gamma = 1.0 + film[0, :, 0, :]
    beta = film[0, :, 1, :]
    src = edge_index[0]
    dst = edge_index[1]
    ea_t = edge_attr.T

    h = x @ p['embed_W'] + p['embed_b']
    for l in range(L):
        cp = p['convs'][l]
        hw_flat = (h @ cp['Wmsg']).T.reshape(-1)
        ew4 = _to_group_layout(cp['Wedge'].T @ ea_t + cp['b'][:, None])
        agg_t = _msgpass(hw_flat, ew4, src, dst).reshape(F, N)
        h = _ln(h + agg_t.T, cp['ln_g'], cp['ln_b'])
        h = gamma[l][None, :] * h + beta[l][None, :]

    W1 = p['head_W1']
    As = h @ W1[:F]
    Bd = h @ W1[F:2 * F]
    C = edge_attr @ W1[2 * F:] + p['head_b1']
    pre = jax.nn.relu(jnp.take(As, src, axis=0) + jnp.take(Bd, dst, axis=0) + C)
    edge_logits = (pre @ p['head_W2'] + p['head_b2'])[:, 0]

    h_graph = jnp.mean(h, axis=0, keepdims=True)
    state = jnp.concatenate([h_flow, h_graph], axis=1)
    value = (jax.nn.relu(state @ p['critic_W1'] + p['critic_b1']) @ p['critic_W2'] + p['critic_b2'])[:, 0]
    return edge_logits, value


# speedup vs baseline: 2.2437x; 1.6970x over previous
import functools

import jax
import jax.numpy as jnp
from jax import lax
from jax.experimental import pallas as pl
from jax.experimental.pallas import tpu as pltpu
from jax.experimental.pallas import tpu_sc as plsc

N = 10000
E = 320000
F = 128
L = 3
NC = 2
NS = 16
NW = NC * NS
LANES = 16
CPT = F // NW
CHUNK = 1280
NCHUNK = E // CHUNK
GROUPS = CHUNK // LANES


def _msgpass_body(hw_ref, ew_ref, src_ref, dst_ref, out_ref,
                  table, agg, src_buf, dst_buf, ew_buf, sem):
    c = lax.axis_index("core")
    s = lax.axis_index("sub")
    w = c * NS + s
    col = w * CPT
    half = w % 2
    grp = w // 2

    pltpu.sync_copy(hw_ref.at[pl.ds(col * N, CPT * N)], table)

    zeros = jnp.zeros((LANES,), jnp.float32)
    @plsc.parallel_loop(0, CPT * N // LANES)
    def _zero(i):
        agg[pl.ds(i * LANES, LANES)] = zeros

    def _start(k, slot):
        base = k * CHUNK
        pltpu.async_copy(src_ref.at[pl.ds(base, CHUNK)], src_buf.at[slot],
                         sem.at[slot])
        pltpu.async_copy(dst_ref.at[pl.ds(base, CHUNK)], dst_buf.at[slot],
                         sem.at[slot])
        pltpu.async_copy(ew_ref.at[k, grp], ew_buf.at[slot], sem.at[slot])

    def _wait(slot):
        pltpu.make_async_copy(src_ref.at[pl.ds(0, CHUNK)], src_buf.at[slot],
                              sem.at[slot]).wait()
        pltpu.make_async_copy(dst_ref.at[pl.ds(0, CHUNK)], dst_buf.at[slot],
                              sem.at[slot]).wait()
        pltpu.make_async_copy(ew_ref.at[0, 0], ew_buf.at[slot],
                              sem.at[slot]).wait()

    def _compute(slot):
        @plsc.parallel_loop(0, GROUPS, unroll=8)
        def _group(g):
            si = src_buf[slot, pl.ds(g * LANES, LANES)]
            di = dst_buf[slot, pl.ds(g * LANES, LANES)]
            for cc in range(CPT):
                gath = plsc.load_gather(table, [si + jnp.int32(cc * N)])
                ew = ew_buf[slot, half * CPT + cc, pl.ds(g * LANES, LANES)]
                m = jnp.maximum(gath + ew, 0.0)
                plsc.addupdate_scatter(agg, [di + jnp.int32(cc * N)], m)

    _start(0, 0)
    def _pair(kk, carry):
        k0 = kk * 2
        _start(k0 + 1, 1)
        _wait(0)
        _compute(0)
        @pl.when(k0 + 2 < NCHUNK)
        def _():
            _start(k0 + 2, 0)
        _wait(1)
        _compute(1)
        return carry
    lax.fori_loop(0, NCHUNK // 2, _pair, 0)

    pltpu.sync_copy(agg, out_ref.at[pl.ds(col * N, CPT * N)])


@jax.jit
def _msgpass(hw_flat, ew4, src, dst):
    mesh = plsc.VectorSubcoreMesh(core_axis_name="core", subcore_axis_name="sub")
    return pl.kernel(
        _msgpass_body,
        out_type=jax.ShapeDtypeStruct((F * N,), jnp.float32),
        mesh=mesh,
        compiler_params=pltpu.CompilerParams(needs_layout_passes=False),
        scratch_types=[
            pltpu.VMEM((CPT * N,), jnp.float32),
            pltpu.VMEM((CPT * N,), jnp.float32),
            pltpu.VMEM((2, CHUNK), jnp.int32),
            pltpu.VMEM((2, CHUNK), jnp.int32),
            pltpu.VMEM((2, 8, CHUNK), jnp.float32),
            pltpu.SemaphoreType.DMA((2,)),
        ],
    )(hw_flat, ew4, src, dst)


def _ln(x, g, b):
    m = jnp.mean(x, axis=-1, keepdims=True)
    v = jnp.var(x, axis=-1, keepdims=True)
    return (x - m) / jnp.sqrt(v + 1e-5) * g + b


def _lstm(xseq, lstm_params):
    h_seq = xseq
    hT = None
    for lp in lstm_params:
        Wih, Whh, b = lp['Wih'], lp['Whh'], lp['b']
        B = h_seq.shape[0]
        Hh = Whh.shape[1]
        def step(carry, xt):
            h, cst = carry
            gates = xt @ Wih.T + h @ Whh.T + b
            i, f, g, o = jnp.split(gates, 4, axis=-1)
            i = jax.nn.sigmoid(i)
            f = jax.nn.sigmoid(f)
            g = jnp.tanh(g)
            o = jax.nn.sigmoid(o)
            cst = f * cst + i * g
            h = o * jnp.tanh(cst)
            return (h, cst), h
        init = (jnp.zeros((B, Hh), jnp.float32), jnp.zeros((B, Hh), jnp.float32))
        (hT, _), ys = lax.scan(step, init, jnp.swapaxes(h_seq, 0, 1))
        h_seq = jnp.swapaxes(ys, 0, 1)
    return hT


def _to_group_layout(ew_t):
    return ew_t.reshape(F // 8, 8, NCHUNK, CHUNK).transpose(2, 0, 1, 3)


def kernel(flow_fingerprint, x, edge_attr, edge_index, batch, params):
    p = params
    h_flow = _lstm(flow_fingerprint, p['lstm'])
    film = jax.nn.relu(h_flow @ p['film_W1'] + p['film_b1']) @ p['film_W2'] + p['film_b2']
    film = film.reshape(-1, L, 2, F)
    gamma = 1.0 + film[0, :, 0, :]
    beta = film[0, :, 1, :]
    src = edge_index[0]
    dst = edge_index[1]
    ea_t = edge_attr.T

    h = x @ p['embed_W'] + p['embed_b']
    for l in range(L):
        cp = p['convs'][l]
        hw_flat = (h @ cp['Wmsg']).T.reshape(-1)
        ew4 = _to_group_layout(cp['Wedge'].T @ ea_t + cp['b'][:, None])
        agg_t = _msgpass(hw_flat, ew4, src, dst).reshape(F, N)
        h = _ln(h + agg_t.T, cp['ln_g'], cp['ln_b'])
        h = gamma[l][None, :] * h + beta[l][None, :]

    W1 = p['head_W1']
    As = h @ W1[:F]
    Bd = h @ W1[F:2 * F]
    C = edge_attr @ W1[2 * F:] + p['head_b1']
    pre = jax.nn.relu(jnp.take(As, src, axis=0) + jnp.take(Bd, dst, axis=0) + C)
    edge_logits = (pre @ p['head_W2'] + p['head_b2'])[:, 0]

    h_graph = jnp.mean(h, axis=0, keepdims=True)
    state = jnp.concatenate([h_flow, h_graph], axis=1)
    value = (jax.nn.relu(state @ p['critic_W1'] + p['critic_b1']) @ p['critic_W2'] + p['critic_b2'])[:, 0]
    return edge_logits, value


# speedup vs baseline: 3.5892x; 1.5997x over previous
import functools

import jax
import jax.numpy as jnp
from jax import lax
from jax.experimental import pallas as pl
from jax.experimental.pallas import tpu as pltpu
from jax.experimental.pallas import tpu_sc as plsc

N = 10000
E = 320000
F = 128
L = 3
NC = 2
NS = 16
NW = NC * NS
LANES = 16
CPT = F // NW
CHUNK = 1280
NCHUNK = E // CHUNK
GROUPS = CHUNK // LANES


def _msgpass_body(hw_ref, ew_ref, src_ref, dst_ref, out_ref,
                  table, agg, src_buf, dst_buf, ew_buf, sem):
    c = lax.axis_index("core")
    s = lax.axis_index("sub")
    w = c * NS + s
    col = w * CPT
    half = w % 2
    grp = w // 2

    pltpu.sync_copy(hw_ref.at[pl.ds(col * N, CPT * N)], table)

    zeros = jnp.zeros((LANES,), jnp.float32)
    @plsc.parallel_loop(0, CPT * N // LANES)
    def _zero(i):
        agg[pl.ds(i * LANES, LANES)] = zeros

    def _start(k, slot):
        base = k * CHUNK
        pltpu.async_copy(src_ref.at[pl.ds(base, CHUNK)], src_buf.at[slot],
                         sem.at[slot])
        pltpu.async_copy(dst_ref.at[pl.ds(base, CHUNK)], dst_buf.at[slot],
                         sem.at[slot])
        pltpu.async_copy(ew_ref.at[k, grp], ew_buf.at[slot], sem.at[slot])

    def _wait(slot):
        pltpu.make_async_copy(src_ref.at[pl.ds(0, CHUNK)], src_buf.at[slot],
                              sem.at[slot]).wait()
        pltpu.make_async_copy(dst_ref.at[pl.ds(0, CHUNK)], dst_buf.at[slot],
                              sem.at[slot]).wait()
        pltpu.make_async_copy(ew_ref.at[0, 0], ew_buf.at[slot],
                              sem.at[slot]).wait()

    def _compute(slot):
        @plsc.parallel_loop(0, GROUPS, unroll=8)
        def _group(g):
            si = src_buf[slot, pl.ds(g * LANES, LANES)]
            di = dst_buf[slot, pl.ds(g * LANES, LANES)]
            for cc in range(CPT):
                gath = plsc.load_gather(table, [si + jnp.int32(cc * N)])
                ew = ew_buf[slot, half * CPT + cc, pl.ds(g * LANES, LANES)]
                m = jnp.maximum(gath + ew, 0.0)
                plsc.addupdate_scatter(agg, [di + jnp.int32(cc * N)], m)

    _start(0, 0)
    def _pair(kk, carry):
        k0 = kk * 2
        _start(k0 + 1, 1)
        _wait(0)
        _compute(0)
        @pl.when(k0 + 2 < NCHUNK)
        def _():
            _start(k0 + 2, 0)
        _wait(1)
        _compute(1)
        return carry
    lax.fori_loop(0, NCHUNK // 2, _pair, 0)

    pltpu.sync_copy(agg, out_ref.at[pl.ds(col * N, CPT * N)])


@jax.jit
def _msgpass(hw_flat, ew4, src, dst):
    mesh = plsc.VectorSubcoreMesh(core_axis_name="core", subcore_axis_name="sub")
    return pl.kernel(
        _msgpass_body,
        out_type=jax.ShapeDtypeStruct((F * N,), jnp.float32),
        mesh=mesh,
        compiler_params=pltpu.CompilerParams(needs_layout_passes=False),
        scratch_types=[
            pltpu.VMEM((CPT * N,), jnp.float32),
            pltpu.VMEM((CPT * N,), jnp.float32),
            pltpu.VMEM((2, CHUNK), jnp.int32),
            pltpu.VMEM((2, CHUNK), jnp.int32),
            pltpu.VMEM((2, 8, CHUNK), jnp.float32),
            pltpu.SemaphoreType.DMA((2,)),
        ],
    )(hw_flat, ew4, src, dst)


def _edgehead_body(as_ref, bd_ref, c_ref, src_ref, dst_ref, w2_ref, out_ref,
                   tab_a, tab_b, src_buf, dst_buf, c_buf, w2_buf, part_buf,
                   sem, sem_o):
    c = lax.axis_index("core")
    s = lax.axis_index("sub")
    w = c * NS + s
    col = w * CPT
    half = w % 2
    grp = w // 2

    pltpu.sync_copy(as_ref.at[pl.ds(col * N, CPT * N)], tab_a)
    pltpu.sync_copy(bd_ref.at[pl.ds(col * N, CPT * N)], tab_b)
    pltpu.sync_copy(w2_ref.at[w], w2_buf)
    w2v = [w2_buf[cc] for cc in range(CPT)]

    def _start(k, slot):
        base = k * CHUNK
        pltpu.async_copy(src_ref.at[pl.ds(base, CHUNK)], src_buf.at[slot],
                         sem.at[slot])
        pltpu.async_copy(dst_ref.at[pl.ds(base, CHUNK)], dst_buf.at[slot],
                         sem.at[slot])
        pltpu.async_copy(c_ref.at[k, grp], c_buf.at[slot], sem.at[slot])

    def _wait(slot):
        pltpu.make_async_copy(src_ref.at[pl.ds(0, CHUNK)], src_buf.at[slot],
                              sem.at[slot]).wait()
        pltpu.make_async_copy(dst_ref.at[pl.ds(0, CHUNK)], dst_buf.at[slot],
                              sem.at[slot]).wait()
        pltpu.make_async_copy(c_ref.at[0, 0], c_buf.at[slot],
                              sem.at[slot]).wait()

    def _wait_out(slot):
        pltpu.make_async_copy(part_buf.at[slot],
                              out_ref.at[pl.ds(0, CHUNK)],
                              sem_o.at[slot]).wait()

    def _compute(slot):
        @plsc.parallel_loop(0, GROUPS, unroll=8)
        def _group(g):
            si = src_buf[slot, pl.ds(g * LANES, LANES)]
            di = dst_buf[slot, pl.ds(g * LANES, LANES)]
            acc = jnp.zeros((LANES,), jnp.float32)
            for cc in range(CPT):
                a = plsc.load_gather(tab_a, [si + jnp.int32(cc * N)])
                b = plsc.load_gather(tab_b, [di + jnp.int32(cc * N)])
                cv = c_buf[slot, half * CPT + cc, pl.ds(g * LANES, LANES)]
                m = jnp.maximum(a + b + cv, 0.0)
                acc = acc + m * w2v[cc]
            part_buf[slot, pl.ds(g * LANES, LANES)] = acc

    def _flush(k, slot):
        pltpu.async_copy(part_buf.at[slot],
                         out_ref.at[pl.ds(w * E + k * CHUNK, CHUNK)],
                         sem_o.at[slot])

    _start(0, 0)
    def _pair(kk, carry):
        k0 = kk * 2
        _start(k0 + 1, 1)
        _wait(0)
        @pl.when(kk > 0)
        def _():
            _wait_out(0)
        _compute(0)
        _flush(k0, 0)
        @pl.when(k0 + 2 < NCHUNK)
        def _():
            _start(k0 + 2, 0)
        _wait(1)
        @pl.when(kk > 0)
        def _():
            _wait_out(1)
        _compute(1)
        _flush(k0 + 1, 1)
        return carry
    lax.fori_loop(0, NCHUNK // 2, _pair, 0)
    _wait_out(0)
    _wait_out(1)


@jax.jit
def _edgehead(as_flat, bd_flat, c4, src, dst, w2g):
    mesh = plsc.VectorSubcoreMesh(core_axis_name="core", subcore_axis_name="sub")
    return pl.kernel(
        _edgehead_body,
        out_type=jax.ShapeDtypeStruct((NW * E,), jnp.float32),
        mesh=mesh,
        compiler_params=pltpu.CompilerParams(needs_layout_passes=False),
        scratch_types=[
            pltpu.VMEM((CPT * N,), jnp.float32),
            pltpu.VMEM((CPT * N,), jnp.float32),
            pltpu.VMEM((2, CHUNK), jnp.int32),
            pltpu.VMEM((2, CHUNK), jnp.int32),
            pltpu.VMEM((2, 8, CHUNK), jnp.float32),
            pltpu.VMEM((CPT, LANES), jnp.float32),
            pltpu.VMEM((2, CHUNK), jnp.float32),
            pltpu.SemaphoreType.DMA((2,)),
            pltpu.SemaphoreType.DMA((2,)),
        ],
    )(as_flat, bd_flat, c4, src, dst, w2g)


def _ln(x, g, b):
    m = jnp.mean(x, axis=-1, keepdims=True)
    v = jnp.var(x, axis=-1, keepdims=True)
    return (x - m) / jnp.sqrt(v + 1e-5) * g + b


def _lstm(xseq, lstm_params):
    h_seq = xseq
    hT = None
    for lp in lstm_params:
        Wih, Whh, b = lp['Wih'], lp['Whh'], lp['b']
        B = h_seq.shape[0]
        Hh = Whh.shape[1]
        def step(carry, xt):
            h, cst = carry
            gates = xt @ Wih.T + h @ Whh.T + b
            i, f, g, o = jnp.split(gates, 4, axis=-1)
            i = jax.nn.sigmoid(i)
            f = jax.nn.sigmoid(f)
            g = jnp.tanh(g)
            o = jax.nn.sigmoid(o)
            cst = f * cst + i * g
            h = o * jnp.tanh(cst)
            return (h, cst), h
        init = (jnp.zeros((B, Hh), jnp.float32), jnp.zeros((B, Hh), jnp.float32))
        (hT, _), ys = lax.scan(step, init, jnp.swapaxes(h_seq, 0, 1))
        h_seq = jnp.swapaxes(ys, 0, 1)
    return hT


def _to_group_layout(ew_t):
    return ew_t.reshape(F // 8, 8, NCHUNK, CHUNK).transpose(2, 0, 1, 3)


def kernel(flow_fingerprint, x, edge_attr, edge_index, batch, params):
    p = params
    h_flow = _lstm(flow_fingerprint, p['lstm'])
    film = jax.nn.relu(h_flow @ p['film_W1'] + p['film_b1']) @ p['film_W2'] + p['film_b2']
    film = film.reshape(-1, L, 2, F)
    gamma = 1.0 + film[0, :, 0, :]
    beta = film[0, :, 1, :]
    src = edge_index[0]
    dst = edge_index[1]
    ea_t = edge_attr.T

    h = x @ p['embed_W'] + p['embed_b']
    for l in range(L):
        cp = p['convs'][l]
        hw_flat = (h @ cp['Wmsg']).T.reshape(-1)
        ew4 = _to_group_layout(cp['Wedge'].T @ ea_t + cp['b'][:, None])
        agg_t = _msgpass(hw_flat, ew4, src, dst).reshape(F, N)
        h = _ln(h + agg_t.T, cp['ln_g'], cp['ln_b'])
        h = gamma[l][None, :] * h + beta[l][None, :]

    W1 = p['head_W1']
    as_flat = (h @ W1[:F]).T.reshape(-1)
    bd_flat = (h @ W1[F:2 * F]).T.reshape(-1)
    c4 = _to_group_layout(W1[2 * F:].T @ ea_t + p['head_b1'][:, None])
    w2 = p['head_W2'][:, 0]
    w2g = jnp.broadcast_to(w2.reshape(NW, CPT)[:, :, None], (NW, CPT, LANES))
    partials = _edgehead(as_flat, bd_flat, c4, src, dst, w2g).reshape(NW, E)
    edge_logits = partials.sum(axis=0) + p['head_b2'][0]

    h_graph = jnp.mean(h, axis=0, keepdims=True)
    state = jnp.concatenate([h_flow, h_graph], axis=1)
    value = (jax.nn.relu(state @ p['critic_W1'] + p['critic_b1']) @ p['critic_W2'] + p['critic_b2'])[:, 0]
    return edge_logits, value


# speedup vs baseline: 4.0904x; 1.1397x over previous
import functools

import jax
import jax.numpy as jnp
from jax import lax
from jax.experimental import pallas as pl
from jax.experimental.pallas import tpu as pltpu
from jax.experimental.pallas import tpu_sc as plsc

N = 10000
E = 320000
F = 128
L = 3
NC = 2
NS = 16
NW = NC * NS
LANES = 16
CPT = F // NW
CHUNK = 1280
NCHUNK = E // CHUNK
GROUPS = CHUNK // LANES


def _msgpass_body(hw_ref, ew_ref, src_ref, dst_ref, out_ref,
                  table, agg, src_buf, dst_buf, ew_buf, sem):
    c = lax.axis_index("core")
    s = lax.axis_index("sub")
    w = c * NS + s
    col = w * CPT
    half = w % 2
    grp = w // 2

    pltpu.sync_copy(hw_ref.at[pl.ds(col * N, CPT * N)], table)

    zeros = jnp.zeros((LANES,), jnp.float32)
    @plsc.parallel_loop(0, CPT * N // LANES)
    def _zero(i):
        agg[pl.ds(i * LANES, LANES)] = zeros

    def _start(k, slot):
        base = k * CHUNK
        pltpu.async_copy(src_ref.at[pl.ds(base, CHUNK)], src_buf.at[slot],
                         sem.at[slot])
        pltpu.async_copy(dst_ref.at[pl.ds(base, CHUNK)], dst_buf.at[slot],
                         sem.at[slot])
        pltpu.async_copy(ew_ref.at[k, grp], ew_buf.at[slot], sem.at[slot])

    def _wait(slot):
        pltpu.make_async_copy(src_ref.at[pl.ds(0, CHUNK)], src_buf.at[slot],
                              sem.at[slot]).wait()
        pltpu.make_async_copy(dst_ref.at[pl.ds(0, CHUNK)], dst_buf.at[slot],
                              sem.at[slot]).wait()
        pltpu.make_async_copy(ew_ref.at[0, 0], ew_buf.at[slot],
                              sem.at[slot]).wait()

    def _compute(slot):
        @plsc.parallel_loop(0, GROUPS, unroll=8)
        def _group(g):
            si = src_buf[slot, pl.ds(g * LANES, LANES)]
            di = dst_buf[slot, pl.ds(g * LANES, LANES)]
            for cc in range(CPT):
                gath = plsc.load_gather(table, [si + jnp.int32(cc * N)])
                ew = ew_buf[slot, half * CPT + cc, pl.ds(g * LANES, LANES)]
                m = jnp.maximum(gath + ew, 0.0)
                plsc.addupdate_scatter(agg, [di + jnp.int32(cc * N)], m)

    _start(0, 0)
    def _pair(kk, carry):
        k0 = kk * 2
        _start(k0 + 1, 1)
        _wait(0)
        _compute(0)
        @pl.when(k0 + 2 < NCHUNK)
        def _():
            _start(k0 + 2, 0)
        _wait(1)
        _compute(1)
        return carry
    lax.fori_loop(0, NCHUNK // 2, _pair, 0)

    pltpu.sync_copy(agg, out_ref.at[pl.ds(col * N, CPT * N)])


@jax.jit
def _msgpass(hw_flat, ew4, src, dst):
    mesh = plsc.VectorSubcoreMesh(core_axis_name="core", subcore_axis_name="sub")
    return pl.kernel(
        _msgpass_body,
        out_type=jax.ShapeDtypeStruct((F * N,), jnp.float32),
        mesh=mesh,
        compiler_params=pltpu.CompilerParams(needs_layout_passes=False),
        scratch_types=[
            pltpu.VMEM((CPT * N,), jnp.float32),
            pltpu.VMEM((CPT * N,), jnp.float32),
            pltpu.VMEM((2, CHUNK), jnp.int32),
            pltpu.VMEM((2, CHUNK), jnp.int32),
            pltpu.VMEM((2, 8, CHUNK), jnp.float32),
            pltpu.SemaphoreType.DMA((2,)),
        ],
    )(hw_flat, ew4, src, dst)


def _edgehead_body(as_ref, bd_ref, c_ref, src_ref, dst_ref, w2_ref, out_ref,
                   tab_a, tab_b, src_buf, dst_buf, c_buf, w2_buf, part_buf,
                   sem, sem_o):
    c = lax.axis_index("core")
    s = lax.axis_index("sub")
    w = c * NS + s
    col = w * CPT
    half = w % 2
    grp = w // 2

    pltpu.sync_copy(as_ref.at[pl.ds(col * N, CPT * N)], tab_a)
    pltpu.sync_copy(bd_ref.at[pl.ds(col * N, CPT * N)], tab_b)
    pltpu.sync_copy(w2_ref.at[w], w2_buf)
    w2v = [w2_buf[cc] for cc in range(CPT)]

    def _start(k, slot):
        base = k * CHUNK
        pltpu.async_copy(src_ref.at[pl.ds(base, CHUNK)], src_buf.at[slot],
                         sem.at[slot])
        pltpu.async_copy(dst_ref.at[pl.ds(base, CHUNK)], dst_buf.at[slot],
                         sem.at[slot])
        pltpu.async_copy(c_ref.at[k, grp], c_buf.at[slot], sem.at[slot])

    def _wait(slot):
        pltpu.make_async_copy(src_ref.at[pl.ds(0, CHUNK)], src_buf.at[slot],
                              sem.at[slot]).wait()
        pltpu.make_async_copy(dst_ref.at[pl.ds(0, CHUNK)], dst_buf.at[slot],
                              sem.at[slot]).wait()
        pltpu.make_async_copy(c_ref.at[0, 0], c_buf.at[slot],
                              sem.at[slot]).wait()

    def _wait_out(slot):
        pltpu.make_async_copy(part_buf.at[slot],
                              out_ref.at[pl.ds(0, CHUNK)],
                              sem_o.at[slot]).wait()

    def _compute(slot):
        @plsc.parallel_loop(0, GROUPS, unroll=8)
        def _group(g):
            si = src_buf[slot, pl.ds(g * LANES, LANES)]
            di = dst_buf[slot, pl.ds(g * LANES, LANES)]
            acc = jnp.zeros((LANES,), jnp.float32)
            for cc in range(CPT):
                a = plsc.load_gather(tab_a, [si + jnp.int32(cc * N)])
                b = plsc.load_gather(tab_b, [di + jnp.int32(cc * N)])
                cv = c_buf[slot, half * CPT + cc, pl.ds(g * LANES, LANES)]
                m = jnp.maximum(a + b + cv, 0.0)
                acc = acc + m * w2v[cc]
            part_buf[slot, pl.ds(g * LANES, LANES)] = acc

    def _flush(k, slot):
        pltpu.async_copy(part_buf.at[slot],
                         out_ref.at[pl.ds(w * E + k * CHUNK, CHUNK)],
                         sem_o.at[slot])

    _start(0, 0)
    def _pair(kk, carry):
        k0 = kk * 2
        _start(k0 + 1, 1)
        _wait(0)
        @pl.when(kk > 0)
        def _():
            _wait_out(0)
        _compute(0)
        _flush(k0, 0)
        @pl.when(k0 + 2 < NCHUNK)
        def _():
            _start(k0 + 2, 0)
        _wait(1)
        @pl.when(kk > 0)
        def _():
            _wait_out(1)
        _compute(1)
        _flush(k0 + 1, 1)
        return carry
    lax.fori_loop(0, NCHUNK // 2, _pair, 0)
    _wait_out(0)
    _wait_out(1)


@jax.jit
def _edgehead(as_flat, bd_flat, c4, src, dst, w2g):
    mesh = plsc.VectorSubcoreMesh(core_axis_name="core", subcore_axis_name="sub")
    return pl.kernel(
        _edgehead_body,
        out_type=jax.ShapeDtypeStruct((NW * E,), jnp.float32),
        mesh=mesh,
        compiler_params=pltpu.CompilerParams(needs_layout_passes=False),
        scratch_types=[
            pltpu.VMEM((CPT * N,), jnp.float32),
            pltpu.VMEM((CPT * N,), jnp.float32),
            pltpu.VMEM((2, CHUNK), jnp.int32),
            pltpu.VMEM((2, CHUNK), jnp.int32),
            pltpu.VMEM((2, 8, CHUNK), jnp.float32),
            pltpu.VMEM((CPT, LANES), jnp.float32),
            pltpu.VMEM((2, CHUNK), jnp.float32),
            pltpu.SemaphoreType.DMA((2,)),
            pltpu.SemaphoreType.DMA((2,)),
        ],
    )(as_flat, bd_flat, c4, src, dst, w2g)


def _groupproj_kernel(w_ref, ea_ref, b_ref, o_ref):
    prod = lax.dot_general(w_ref[...], ea_ref[...], (((0,), (0,)), ((), ())),
                           preferred_element_type=jnp.float32)
    prod = prod + b_ref[...]
    for j in range(F // 8):
        o_ref[0, j] = prod[j * 8:(j + 1) * 8, :]


@jax.jit
def _groupproj(w, ea_t, b):
    return pl.pallas_call(
        _groupproj_kernel,
        grid=(NCHUNK,),
        in_specs=[
            pl.BlockSpec((w.shape[0], F), lambda k: (0, 0)),
            pl.BlockSpec((ea_t.shape[0], CHUNK), lambda k: (0, k)),
            pl.BlockSpec((F, 1), lambda k: (0, 0)),
        ],
        out_specs=pl.BlockSpec((1, F // 8, 8, CHUNK), lambda k: (k, 0, 0, 0)),
        out_shape=jax.ShapeDtypeStruct((NCHUNK, F // 8, 8, CHUNK), jnp.float32),
    )(w, ea_t, b)


def _ln(x, g, b):
    m = jnp.mean(x, axis=-1, keepdims=True)
    v = jnp.var(x, axis=-1, keepdims=True)
    return (x - m) / jnp.sqrt(v + 1e-5) * g + b


def _lstm(xseq, lstm_params):
    h_seq = xseq
    hT = None
    for lp in lstm_params:
        Wih, Whh, b = lp['Wih'], lp['Whh'], lp['b']
        B = h_seq.shape[0]
        Hh = Whh.shape[1]
        def step(carry, xt):
            h, cst = carry
            gates = xt @ Wih.T + h @ Whh.T + b
            i, f, g, o = jnp.split(gates, 4, axis=-1)
            i = jax.nn.sigmoid(i)
            f = jax.nn.sigmoid(f)
            g = jnp.tanh(g)
            o = jax.nn.sigmoid(o)
            cst = f * cst + i * g
            h = o * jnp.tanh(cst)
            return (h, cst), h
        init = (jnp.zeros((B, Hh), jnp.float32), jnp.zeros((B, Hh), jnp.float32))
        (hT, _), ys = lax.scan(step, init, jnp.swapaxes(h_seq, 0, 1))
        h_seq = jnp.swapaxes(ys, 0, 1)
    return hT


def kernel(flow_fingerprint, x, edge_attr, edge_index, batch, params):
    p = params
    h_flow = _lstm(flow_fingerprint, p['lstm'])
    film = jax.nn.relu(h_flow @ p['film_W1'] + p['film_b1']) @ p['film_W2'] + p['film_b2']
    film = film.reshape(-1, L, 2, F)
    gamma = 1.0 + film[0, :, 0, :]
    beta = film[0, :, 1, :]
    src = edge_index[0]
    dst = edge_index[1]
    ea_t = edge_attr.T

    h = x @ p['embed_W'] + p['embed_b']
    for l in range(L):
        cp = p['convs'][l]
        hw_flat = (h @ cp['Wmsg']).T.reshape(-1)
        ew4 = _groupproj(cp['Wedge'], ea_t, cp['b'][:, None])
        agg_t = _msgpass(hw_flat, ew4, src, dst).reshape(F, N)
        h = _ln(h + agg_t.T, cp['ln_g'], cp['ln_b'])
        h = gamma[l][None, :] * h + beta[l][None, :]

    W1 = p['head_W1']
    as_flat = (h @ W1[:F]).T.reshape(-1)
    bd_flat = (h @ W1[F:2 * F]).T.reshape(-1)
    c4 = _groupproj(W1[2 * F:], ea_t, p['head_b1'][:, None])
    w2 = p['head_W2'][:, 0]
    w2g = jnp.broadcast_to(w2.reshape(NW, CPT)[:, :, None], (NW, CPT, LANES))
    partials = _edgehead(as_flat, bd_flat, c4, src, dst, w2g).reshape(NW, E)
    edge_logits = partials.sum(axis=0) + p['head_b2'][0]

    h_graph = jnp.mean(h, axis=0, keepdims=True)
    state = jnp.concatenate([h_flow, h_graph], axis=1)
    value = (jax.nn.relu(state @ p['critic_W1'] + p['critic_b1']) @ p['critic_W2'] + p['critic_b2'])[:, 0]
    return edge_logits, value


# speedup vs baseline: 4.5322x; 1.1080x over previous
import functools

import jax
import jax.numpy as jnp
from jax import lax
from jax.experimental import pallas as pl
from jax.experimental.pallas import tpu as pltpu
from jax.experimental.pallas import tpu_sc as plsc

N = 10000
E = 320000
F = 128
L = 3
NC = 2
NS = 16
NW = NC * NS
LANES = 16
CPT = F // NW
CPK = CPT // 2
CHUNK = 3200
NCHUNK = E // CHUNK
GROUPS = CHUNK // LANES
UNROLL = 10


def _unpack_pair(gi):
    gbf = plsc.bitcast(gi, jnp.bfloat16)
    return plsc.unpack(gbf, format=plsc.PackFormat.INTERLEAVED)


def _msgpass_body(hw_ref, ew_ref, idx_ref, out_ref,
                  table, agg, idx_buf, ew_buf, sem):
    c = lax.axis_index("core")
    s = lax.axis_index("sub")
    w = c * NS + s
    col = w * CPT
    half = w % 2
    grp = w // 2

    pltpu.sync_copy(hw_ref.at[pl.ds(w * CPK * N, CPK * N)], table)

    zeros = jnp.zeros((LANES,), jnp.float32)
    @plsc.parallel_loop(0, CPT * N // LANES)
    def _zero(i):
        agg[pl.ds(i * LANES, LANES)] = zeros

    def _start(k, slot):
        pltpu.async_copy(idx_ref.at[pl.ds(k * CHUNK, CHUNK)], idx_buf.at[slot],
                         sem.at[slot])
        pltpu.async_copy(ew_ref.at[k, grp], ew_buf.at[slot], sem.at[slot])

    def _wait(slot):
        pltpu.make_async_copy(idx_ref.at[pl.ds(0, CHUNK)], idx_buf.at[slot],
                              sem.at[slot]).wait()
        pltpu.make_async_copy(ew_ref.at[0, 0], ew_buf.at[slot],
                              sem.at[slot]).wait()

    def _compute(slot):
        @plsc.parallel_loop(0, GROUPS, unroll=UNROLL)
        def _group(g):
            p = idx_buf[slot, pl.ds(g * LANES, LANES)]
            si = p & jnp.int32(16383)
            di = p >> 14
            for cc2 in range(CPK):
                gi = plsc.load_gather(table, [si + jnp.int32(cc2 * N)])
                lo, hi = _unpack_pair(gi)
                for sub, gval in ((0, lo), (1, hi)):
                    cc = cc2 * 2 + sub
                    ew = ew_buf[slot, half * CPT + cc, pl.ds(g * LANES, LANES)]
                    m = jnp.maximum(gval + ew, 0.0)
                    plsc.addupdate_scatter(agg, [di + jnp.int32(cc * N)], m)

    _start(0, 0)
    def _pair(kk, carry):
        k0 = kk * 2
        _start(k0 + 1, 1)
        _wait(0)
        _compute(0)
        @pl.when(k0 + 2 < NCHUNK)
        def _():
            _start(k0 + 2, 0)
        _wait(1)
        _compute(1)
        return carry
    lax.fori_loop(0, NCHUNK // 2, _pair, 0)

    pltpu.sync_copy(agg, out_ref.at[pl.ds(col * N, CPT * N)])


@jax.jit
def _msgpass(hwp_flat, ew4, pidx):
    mesh = plsc.VectorSubcoreMesh(core_axis_name="core", subcore_axis_name="sub")
    return pl.kernel(
        _msgpass_body,
        out_type=jax.ShapeDtypeStruct((F * N,), jnp.float32),
        mesh=mesh,
        compiler_params=pltpu.CompilerParams(needs_layout_passes=False),
        scratch_types=[
            pltpu.VMEM((CPK * N,), jnp.int32),
            pltpu.VMEM((CPT * N,), jnp.float32),
            pltpu.VMEM((2, CHUNK), jnp.int32),
            pltpu.VMEM((2, 8, CHUNK), jnp.float32),
            pltpu.SemaphoreType.DMA((2,)),
        ],
    )(hwp_flat, ew4, pidx)


def _edgehead_body(as_ref, bd_ref, c_ref, idx_ref, w2_ref, out_ref,
                   tab_a, tab_b, idx_buf, c_buf, w2_buf, part_buf,
                   sem, sem_o):
    c = lax.axis_index("core")
    s = lax.axis_index("sub")
    w = c * NS + s
    half = w % 2
    grp = w // 2

    pltpu.sync_copy(as_ref.at[pl.ds(w * CPK * N, CPK * N)], tab_a)
    pltpu.sync_copy(bd_ref.at[pl.ds(w * CPK * N, CPK * N)], tab_b)
    pltpu.sync_copy(w2_ref.at[w], w2_buf)
    w2v = [w2_buf[cc] for cc in range(CPT)]

    def _start(k, slot):
        pltpu.async_copy(idx_ref.at[pl.ds(k * CHUNK, CHUNK)], idx_buf.at[slot],
                         sem.at[slot])
        pltpu.async_copy(c_ref.at[k, grp], c_buf.at[slot], sem.at[slot])

    def _wait(slot):
        pltpu.make_async_copy(idx_ref.at[pl.ds(0, CHUNK)], idx_buf.at[slot],
                              sem.at[slot]).wait()
        pltpu.make_async_copy(c_ref.at[0, 0], c_buf.at[slot],
                              sem.at[slot]).wait()

    def _wait_out(slot):
        pltpu.make_async_copy(part_buf.at[slot],
                              out_ref.at[pl.ds(0, CHUNK)],
                              sem_o.at[slot]).wait()

    def _compute(slot):
        @plsc.parallel_loop(0, GROUPS, unroll=UNROLL)
        def _group(g):
            p = idx_buf[slot, pl.ds(g * LANES, LANES)]
            si = p & jnp.int32(16383)
            di = p >> 14
            acc = jnp.zeros((LANES,), jnp.float32)
            for cc2 in range(CPK):
                ga = plsc.load_gather(tab_a, [si + jnp.int32(cc2 * N)])
                gb = plsc.load_gather(tab_b, [di + jnp.int32(cc2 * N)])
                alo, ahi = _unpack_pair(ga)
                blo, bhi = _unpack_pair(gb)
                for sub, (av, bv) in ((0, (alo, blo)), (1, (ahi, bhi))):
                    cc = cc2 * 2 + sub
                    cv = c_buf[slot, half * CPT + cc, pl.ds(g * LANES, LANES)]
                    m = jnp.maximum(av + bv + cv, 0.0)
                    acc = acc + m * w2v[cc]
            part_buf[slot, pl.ds(g * LANES, LANES)] = acc

    def _flush(k, slot):
        pltpu.async_copy(part_buf.at[slot],
                         out_ref.at[pl.ds(w * E + k * CHUNK, CHUNK)],
                         sem_o.at[slot])

    _start(0, 0)
    def _pair(kk, carry):
        k0 = kk * 2
        _start(k0 + 1, 1)
        _wait(0)
        @pl.when(kk > 0)
        def _():
            _wait_out(0)
        _compute(0)
        _flush(k0, 0)
        @pl.when(k0 + 2 < NCHUNK)
        def _():
            _start(k0 + 2, 0)
        _wait(1)
        @pl.when(kk > 0)
        def _():
            _wait_out(1)
        _compute(1)
        _flush(k0 + 1, 1)
        return carry
    lax.fori_loop(0, NCHUNK // 2, _pair, 0)
    _wait_out(0)
    _wait_out(1)


@jax.jit
def _edgehead(as_pk, bd_pk, c4, pidx, w2g):
    mesh = plsc.VectorSubcoreMesh(core_axis_name="core", subcore_axis_name="sub")
    return pl.kernel(
        _edgehead_body,
        out_type=jax.ShapeDtypeStruct((NW * E,), jnp.float32),
        mesh=mesh,
        compiler_params=pltpu.CompilerParams(needs_layout_passes=False),
        scratch_types=[
            pltpu.VMEM((CPK * N,), jnp.int32),
            pltpu.VMEM((CPK * N,), jnp.int32),
            pltpu.VMEM((2, CHUNK), jnp.int32),
            pltpu.VMEM((2, 8, CHUNK), jnp.float32),
            pltpu.VMEM((CPT, LANES), jnp.float32),
            pltpu.VMEM((2, CHUNK), jnp.float32),
            pltpu.SemaphoreType.DMA((2,)),
            pltpu.SemaphoreType.DMA((2,)),
        ],
    )(as_pk, bd_pk, c4, pidx, w2g)


def _groupproj_kernel(w_ref, ea_ref, b_ref, o_ref):
    prod = lax.dot_general(w_ref[...], ea_ref[...], (((0,), (0,)), ((), ())),
                           preferred_element_type=jnp.float32)
    prod = prod + b_ref[...]
    for j in range(F // 8):
        o_ref[0, j] = prod[j * 8:(j + 1) * 8, :]


@jax.jit
def _groupproj(w, ea_t, b):
    return pl.pallas_call(
        _groupproj_kernel,
        grid=(NCHUNK,),
        in_specs=[
            pl.BlockSpec((w.shape[0], F), lambda k: (0, 0)),
            pl.BlockSpec((ea_t.shape[0], CHUNK), lambda k: (0, k)),
            pl.BlockSpec((F, 1), lambda k: (0, 0)),
        ],
        out_specs=pl.BlockSpec((1, F // 8, 8, CHUNK), lambda k: (k, 0, 0, 0)),
        out_shape=jax.ShapeDtypeStruct((NCHUNK, F // 8, 8, CHUNK), jnp.float32),
    )(w, ea_t, b)


def _pack_table(hw):
    hw_bf = hw.astype(jnp.bfloat16)
    pk = jax.lax.bitcast_convert_type(hw_bf.reshape(N, F // 2, 2), jnp.int32)
    return pk.T.reshape(-1)


def _ln(x, g, b):
    m = jnp.mean(x, axis=-1, keepdims=True)
    v = jnp.var(x, axis=-1, keepdims=True)
    return (x - m) / jnp.sqrt(v + 1e-5) * g + b


def _lstm(xseq, lstm_params):
    h_seq = xseq
    hT = None
    for lp in lstm_params:
        Wih, Whh, b = lp['Wih'], lp['Whh'], lp['b']
        B = h_seq.shape[0]
        Hh = Whh.shape[1]
        def step(carry, xt):
            h, cst = carry
            gates = xt @ Wih.T + h @ Whh.T + b
            i, f, g, o = jnp.split(gates, 4, axis=-1)
            i = jax.nn.sigmoid(i)
            f = jax.nn.sigmoid(f)
            g = jnp.tanh(g)
            o = jax.nn.sigmoid(o)
            cst = f * cst + i * g
            h = o * jnp.tanh(cst)
            return (h, cst), h
        init = (jnp.zeros((B, Hh), jnp.float32), jnp.zeros((B, Hh), jnp.float32))
        (hT, _), ys = lax.scan(step, init, jnp.swapaxes(h_seq, 0, 1))
        h_seq = jnp.swapaxes(ys, 0, 1)
    return hT


def kernel(flow_fingerprint, x, edge_attr, edge_index, batch, params):
    p = params
    h_flow = _lstm(flow_fingerprint, p['lstm'])
    film = jax.nn.relu(h_flow @ p['film_W1'] + p['film_b1']) @ p['film_W2'] + p['film_b2']
    film = film.reshape(-1, L, 2, F)
    gamma = 1.0 + film[0, :, 0, :]
    beta = film[0, :, 1, :]
    src = edge_index[0]
    dst = edge_index[1]
    pidx = src | (dst << 14)
    ea_t = edge_attr.T

    h = x @ p['embed_W'] + p['embed_b']
    for l in range(L):
        cp = p['convs'][l]
        hwp = _pack_table(h @ cp['Wmsg'])
        ew4 = _groupproj(cp['Wedge'], ea_t, cp['b'][:, None])
        agg_t = _msgpass(hwp, ew4, pidx).reshape(F, N)
        h = _ln(h + agg_t.T, cp['ln_g'], cp['ln_b'])
        h = gamma[l][None, :] * h + beta[l][None, :]

    W1 = p['head_W1']
    as_pk = _pack_table(h @ W1[:F])
    bd_pk = _pack_table(h @ W1[F:2 * F])
    c4 = _groupproj(W1[2 * F:], ea_t, p['head_b1'][:, None])
    w2 = p['head_W2'][:, 0]
    w2g = jnp.broadcast_to(w2.reshape(NW, CPT)[:, :, None], (NW, CPT, LANES))
    partials = _edgehead(as_pk, bd_pk, c4, pidx, w2g).reshape(NW, E)
    edge_logits = partials.sum(axis=0) + p['head_b2'][0]

    h_graph = jnp.mean(h, axis=0, keepdims=True)
    state = jnp.concatenate([h_flow, h_graph], axis=1)
    value = (jax.nn.relu(state @ p['critic_W1'] + p['critic_b1']) @ p['critic_W2'] + p['critic_b2'])[:, 0]
    return edge_logits, value
